# Initial kernel scaffold; baseline (speedup 1.0000x reference)
import jax
import jax.numpy as jnp
from jax.experimental import pallas as pl


def kernel(device_idx, matrix, features, W_ih_f, W_hh_f, b_ih_f, b_hh_f, W_ih_b, W_hh_b, b_ih_b, b_hh_b, W_fc, b_fc):
    raise NotImplementedError("write your pallas kernel here")


# speedup vs baseline: 2.8561x; 2.8561x over previous
import jax
import jax.numpy as jnp
from jax.experimental import pallas as pl
from jax.experimental.pallas import tpu as pltpu

B, N, IN, H = 16, 512, 6, 256
G = 4 * H
INP = 8


def _fused_kernel(dev_ref, mat_ref, xaug_ref,
                  wih_f_ref, whh_f_ref, bihf_ref, bhhf_ref,
                  wih_b_ref, whh_b_ref, bihb_ref, bhhb_ref,
                  wfc_h_ref, wfc_d_ref, bfc_ref,
                  out_ref, agg_ref):
    for b in range(B):
        mask = (mat_ref[b] > 0).astype(jnp.float32)
        xb = xaug_ref[b]
        s = jnp.dot(mask, xb, preferred_element_type=jnp.float32)
        tot = xb + s
        inv = 1.0 / tot[:, 6:7]
        agg_ref[:, b, :] = tot * inv

    wih_f = wih_f_ref[...]
    whh_f = whh_f_ref[...]
    bias_f = bihf_ref[...] + bhhf_ref[...]
    wih_b = wih_b_ref[...]
    whh_b = whh_b_ref[...]
    bias_b = bihb_ref[...] + bhhb_ref[...]

    def gates(x_t, h, c, wih, whh, bias):
        g = (jnp.dot(x_t, wih, preferred_element_type=jnp.float32)
             + jnp.dot(h, whh, preferred_element_type=jnp.float32)
             + bias)
        i = jax.nn.sigmoid(g[:, :H])
        f = jax.nn.sigmoid(g[:, H:2 * H])
        gg = jnp.tanh(g[:, 2 * H:3 * H])
        o = jax.nn.sigmoid(g[:, 3 * H:])
        c2 = f * c + i * gg
        h2 = o * jnp.tanh(c2)
        return h2, c2

    def step(t, carry):
        h_f, c_f, h_b, c_b = carry
        x_f = agg_ref[t]
        x_b = agg_ref[N - 1 - t]
        h_f, c_f = gates(x_f, h_f, c_f, wih_f, whh_f, bias_f)
        h_b, c_b = gates(x_b, h_b, c_b, wih_b, whh_b, bias_b)
        return h_f, c_f, h_b, c_b

    z = jnp.zeros((B, H), jnp.float32)
    h_f, c_f, h_b, c_b = jax.lax.fori_loop(0, N, step, (z, z, z, z))

    hcat = jnp.concatenate([h_f, h_b], axis=1)
    out_ref[...] = (jnp.dot(hcat, wfc_h_ref[...],
                            preferred_element_type=jnp.float32)
                    + dev_ref[...] * wfc_d_ref[...] + bfc_ref[...])


def kernel(device_idx, matrix, features, W_ih_f, W_hh_f, b_ih_f, b_hh_f,
           W_ih_b, W_hh_b, b_ih_b, b_hh_b, W_fc, b_fc):
    x = features.astype(jnp.float32)
    pad = jnp.concatenate(
        [jnp.ones((B, N, 1), jnp.float32), jnp.zeros((B, N, 1), jnp.float32)],
        axis=2)
    xaug = jnp.concatenate([x, pad], axis=2)

    wih_f = jnp.pad(W_ih_f.T, ((0, INP - IN), (0, 0)))
    wih_b = jnp.pad(W_ih_b.T, ((0, INP - IN), (0, 0)))
    whh_f = W_hh_f.T
    whh_b = W_hh_b.T

    out = pl.pallas_call(
        _fused_kernel,
        out_shape=jax.ShapeDtypeStruct((B, 1), jnp.float32),
        scratch_shapes=[pltpu.VMEM((N, B, INP), jnp.float32)],
    )(device_idx.reshape(B, 1), matrix, xaug,
      wih_f, whh_f, b_ih_f.reshape(1, G), b_hh_f.reshape(1, G),
      wih_b, whh_b, b_ih_b.reshape(1, G), b_hh_b.reshape(1, G),
      W_fc[:, 1:].T, W_fc[:, :1], b_fc.reshape(1, 1))
    return out.reshape(-1)


# speedup vs baseline: 3.0157x; 1.0559x over previous
import jax
import jax.numpy as jnp
from jax.experimental import pallas as pl
from jax.experimental.pallas import tpu as pltpu

B, N, IN, H = 16, 512, 6, 256
G = 4 * H
INP = 8
CH = 128


def _fused_kernel(dev_ref, mat_ref, xaug_ref,
                  wih_f_ref, whh_f_ref, bihf_ref, bhhf_ref,
                  wih_b_ref, whh_b_ref, bihb_ref, bhhb_ref,
                  wfc_h_ref, wfc_d_ref, bfc_ref,
                  out_ref, agg_ref, xwf_ref, xwb_ref):
    for b in range(B):
        mask = (mat_ref[b] > 0).astype(jnp.float32)
        xb = xaug_ref[b]
        s = jnp.dot(mask, xb, preferred_element_type=jnp.float32)
        tot = xb + s
        inv = 1.0 / tot[:, 6:7]
        agg_ref[:, b, :] = tot * inv

    wih_f = wih_f_ref[...]
    whh_f = whh_f_ref[...]
    bias_f = bihf_ref[...] + bhhf_ref[...]
    wih_b = wih_b_ref[...]
    whh_b = whh_b_ref[...]
    bias_b = bihb_ref[...] + bhhb_ref[...]

    def lstm_cell(g, c):
        i = jax.nn.sigmoid(g[:, :H])
        f = jax.nn.sigmoid(g[:, H:2 * H])
        gg = jnp.tanh(g[:, 2 * H:3 * H])
        o = jax.nn.sigmoid(g[:, 3 * H:])
        c2 = f * c + i * gg
        h2 = o * jnp.tanh(c2)
        return h2, c2

    def step(i, carry):
        h_f, c_f, h_b, c_b = carry
        g_f = xwf_ref[i] + jnp.dot(h_f, whh_f,
                                   preferred_element_type=jnp.float32)
        g_b = xwb_ref[CH - 1 - i] + jnp.dot(h_b, whh_b,
                                            preferred_element_type=jnp.float32)
        h_f, c_f = lstm_cell(g_f, c_f)
        h_b, c_b = lstm_cell(g_b, c_b)
        return h_f, c_f, h_b, c_b

    z = jnp.zeros((B, H), jnp.float32)
    carry = (z, z, z, z)
    for c in range(N // CH):
        af = agg_ref[pl.ds(c * CH, CH)].reshape(CH * B, INP)
        ab = agg_ref[pl.ds(N - (c + 1) * CH, CH)].reshape(CH * B, INP)
        xwf_ref[...] = (jnp.dot(af, wih_f, preferred_element_type=jnp.float32)
                        + bias_f).reshape(CH, B, G)
        xwb_ref[...] = (jnp.dot(ab, wih_b, preferred_element_type=jnp.float32)
                        + bias_b).reshape(CH, B, G)
        carry = jax.lax.fori_loop(0, CH, step, carry)
    h_f, c_f, h_b, c_b = carry

    hcat = jnp.concatenate([h_f, h_b], axis=1)
    out_ref[...] = (jnp.dot(hcat, wfc_h_ref[...],
                            preferred_element_type=jnp.float32)
                    + dev_ref[...] * wfc_d_ref[...] + bfc_ref[...])


def kernel(device_idx, matrix, features, W_ih_f, W_hh_f, b_ih_f, b_hh_f,
           W_ih_b, W_hh_b, b_ih_b, b_hh_b, W_fc, b_fc):
    x = features.astype(jnp.float32)
    pad = jnp.concatenate(
        [jnp.ones((B, N, 1), jnp.float32), jnp.zeros((B, N, 1), jnp.float32)],
        axis=2)
    xaug = jnp.concatenate([x, pad], axis=2)

    wih_f = jnp.pad(W_ih_f.T, ((0, INP - IN), (0, 0)))
    wih_b = jnp.pad(W_ih_b.T, ((0, INP - IN), (0, 0)))
    whh_f = W_hh_f.T
    whh_b = W_hh_b.T

    out = pl.pallas_call(
        _fused_kernel,
        out_shape=jax.ShapeDtypeStruct((B, 1), jnp.float32),
        scratch_shapes=[pltpu.VMEM((N, B, INP), jnp.float32),
                        pltpu.VMEM((CH, B, G), jnp.float32),
                        pltpu.VMEM((CH, B, G), jnp.float32)],
    )(device_idx.reshape(B, 1), matrix, xaug,
      wih_f, whh_f, b_ih_f.reshape(1, G), b_hh_f.reshape(1, G),
      wih_b, whh_b, b_ih_b.reshape(1, G), b_hh_b.reshape(1, G),
      W_fc[:, 1:].T, W_fc[:, :1], b_fc.reshape(1, 1))
    return out.reshape(-1)


# speedup vs baseline: 3.0203x; 1.0015x over previous
import jax
import jax.numpy as jnp
from jax.experimental import pallas as pl
from jax.experimental.pallas import tpu as pltpu

B, N, IN, H = 16, 512, 6, 256
G = 4 * H
INP = 8
CH = 128


def _fused_kernel(dev_ref, mat_ref, xaug_ref,
                  wih_f_ref, whh_f_ref, bihf_ref, bhhf_ref,
                  wih_b_ref, whh_b_ref, bihb_ref, bhhb_ref,
                  wfc_h_ref, wfc_d_ref, bfc_ref,
                  out_ref, agg_ref, xwf_ref, xwb_ref):
    for b in range(B):
        mask = (mat_ref[b] > 0).astype(jnp.float32)
        xb = xaug_ref[b]
        s = jnp.dot(mask, xb, preferred_element_type=jnp.float32)
        tot = xb + s
        inv = 1.0 / tot[:, 6:7]
        agg_ref[:, b, :] = tot * inv

    wih_f = wih_f_ref[...]
    whh_f = whh_f_ref[...]
    bias_f = bihf_ref[...] + bhhf_ref[...]
    wih_b = wih_b_ref[...]
    whh_b = whh_b_ref[...]
    bias_b = bihb_ref[...] + bhhb_ref[...]

    def lstm_cell(g, c):
        i = jax.nn.sigmoid(g[:, :H])
        f = jax.nn.sigmoid(g[:, H:2 * H])
        gg = jnp.tanh(g[:, 2 * H:3 * H])
        o = jax.nn.sigmoid(g[:, 3 * H:])
        c2 = f * c + i * gg
        h2 = o * jnp.tanh(c2)
        return h2, c2

    whh_f16 = whh_f.astype(jnp.bfloat16)
    whh_b16 = whh_b.astype(jnp.bfloat16)

    def step(i, carry):
        h_f, c_f, h_b, c_b = carry
        g_f = xwf_ref[i] + jnp.dot(h_f.astype(jnp.bfloat16), whh_f16,
                                   preferred_element_type=jnp.float32)
        g_b = xwb_ref[CH - 1 - i] + jnp.dot(h_b.astype(jnp.bfloat16), whh_b16,
                                            preferred_element_type=jnp.float32)
        h_f, c_f = lstm_cell(g_f, c_f)
        h_b, c_b = lstm_cell(g_b, c_b)
        return h_f, c_f, h_b, c_b

    z = jnp.zeros((B, H), jnp.float32)
    carry = (z, z, z, z)
    for c in range(N // CH):
        af = agg_ref[pl.ds(c * CH, CH)].reshape(CH * B, INP)
        ab = agg_ref[pl.ds(N - (c + 1) * CH, CH)].reshape(CH * B, INP)
        xwf_ref[...] = (jnp.dot(af, wih_f, preferred_element_type=jnp.float32)
                        + bias_f).reshape(CH, B, G)
        xwb_ref[...] = (jnp.dot(ab, wih_b, preferred_element_type=jnp.float32)
                        + bias_b).reshape(CH, B, G)
        carry = jax.lax.fori_loop(0, CH, step, carry)
    h_f, c_f, h_b, c_b = carry

    hcat = jnp.concatenate([h_f, h_b], axis=1)
    out_ref[...] = (jnp.dot(hcat, wfc_h_ref[...],
                            preferred_element_type=jnp.float32)
                    + dev_ref[...] * wfc_d_ref[...] + bfc_ref[...])


def kernel(device_idx, matrix, features, W_ih_f, W_hh_f, b_ih_f, b_hh_f,
           W_ih_b, W_hh_b, b_ih_b, b_hh_b, W_fc, b_fc):
    x = features.astype(jnp.float32)
    pad = jnp.concatenate(
        [jnp.ones((B, N, 1), jnp.float32), jnp.zeros((B, N, 1), jnp.float32)],
        axis=2)
    xaug = jnp.concatenate([x, pad], axis=2)

    wih_f = jnp.pad(W_ih_f.T, ((0, INP - IN), (0, 0)))
    wih_b = jnp.pad(W_ih_b.T, ((0, INP - IN), (0, 0)))
    whh_f = W_hh_f.T
    whh_b = W_hh_b.T

    out = pl.pallas_call(
        _fused_kernel,
        out_shape=jax.ShapeDtypeStruct((B, 1), jnp.float32),
        scratch_shapes=[pltpu.VMEM((N, B, INP), jnp.float32),
                        pltpu.VMEM((CH, B, G), jnp.float32),
                        pltpu.VMEM((CH, B, G), jnp.float32)],
    )(device_idx.reshape(B, 1), matrix, xaug,
      wih_f, whh_f, b_ih_f.reshape(1, G), b_hh_f.reshape(1, G),
      wih_b, whh_b, b_ih_b.reshape(1, G), b_hh_b.reshape(1, G),
      W_fc[:, 1:].T, W_fc[:, :1], b_fc.reshape(1, 1))
    return out.reshape(-1)


# speedup vs baseline: 3.5316x; 1.1693x over previous
import jax
import jax.numpy as jnp
from jax.experimental import pallas as pl
from jax.experimental.pallas import tpu as pltpu

B, N, IN, H = 16, 512, 6, 256
G = 4 * H
INP = 8
CH = 128


def _fused_kernel(dev_ref, mat_ref, xaug_ref,
                  wih_f_ref, whh_f_ref, bihf_ref, bhhf_ref,
                  wih_b_ref, whh_b_ref, bihb_ref, bhhb_ref,
                  wfc_h_ref, wfc_d_ref, bfc_ref,
                  out_ref, agg_ref, xwf_ref, xwb_ref):
    for b in range(B):
        mask = (mat_ref[b] > 0).astype(jnp.float32)
        xb = xaug_ref[b]
        s = jnp.dot(mask, xb, preferred_element_type=jnp.float32)
        tot = xb + s
        inv = 1.0 / tot[:, 6:7]
        agg_ref[:, b, :] = tot * inv

    wih_f = wih_f_ref[...]
    whh_f = whh_f_ref[...]
    bias_f = bihf_ref[...] + bhhf_ref[...]
    wih_b = wih_b_ref[...]
    whh_b = whh_b_ref[...]
    bias_b = bihb_ref[...] + bhhb_ref[...]

    def lstm_cell(g, c):
        i = jax.nn.sigmoid(g[:, :H])
        f = jax.nn.sigmoid(g[:, H:2 * H])
        gg = jnp.tanh(g[:, 2 * H:3 * H])
        o = jax.nn.sigmoid(g[:, 3 * H:])
        c2 = f * c + i * gg
        h2 = o * jnp.tanh(c2)
        return h2, c2

    whh_f16 = whh_f.astype(jnp.bfloat16)
    whh_b16 = whh_b.astype(jnp.bfloat16)

    def one_step(i, carry):
        h_f, c_f, h_b, c_b = carry
        g_f = xwf_ref[i] + jnp.dot(h_f.astype(jnp.bfloat16), whh_f16,
                                   preferred_element_type=jnp.float32)
        g_b = xwb_ref[CH - 1 - i] + jnp.dot(h_b.astype(jnp.bfloat16), whh_b16,
                                            preferred_element_type=jnp.float32)
        h_f, c_f = lstm_cell(g_f, c_f)
        h_b, c_b = lstm_cell(g_b, c_b)
        return h_f, c_f, h_b, c_b

    def step(j, carry):
        carry = one_step(2 * j, carry)
        return one_step(2 * j + 1, carry)

    z = jnp.zeros((B, H), jnp.float32)
    carry = (z, z, z, z)
    for c in range(N // CH):
        af = agg_ref[pl.ds(c * CH, CH)].reshape(CH * B, INP)
        ab = agg_ref[pl.ds(N - (c + 1) * CH, CH)].reshape(CH * B, INP)
        xwf_ref[...] = (jnp.dot(af, wih_f, preferred_element_type=jnp.float32)
                        + bias_f).reshape(CH, B, G)
        xwb_ref[...] = (jnp.dot(ab, wih_b, preferred_element_type=jnp.float32)
                        + bias_b).reshape(CH, B, G)
        carry = jax.lax.fori_loop(0, CH // 2, step, carry)
    h_f, c_f, h_b, c_b = carry

    hcat = jnp.concatenate([h_f, h_b], axis=1)
    out_ref[...] = (jnp.dot(hcat, wfc_h_ref[...],
                            preferred_element_type=jnp.float32)
                    + dev_ref[...] * wfc_d_ref[...] + bfc_ref[...])


def kernel(device_idx, matrix, features, W_ih_f, W_hh_f, b_ih_f, b_hh_f,
           W_ih_b, W_hh_b, b_ih_b, b_hh_b, W_fc, b_fc):
    x = features.astype(jnp.float32)
    pad = jnp.concatenate(
        [jnp.ones((B, N, 1), jnp.float32), jnp.zeros((B, N, 1), jnp.float32)],
        axis=2)
    xaug = jnp.concatenate([x, pad], axis=2)

    wih_f = jnp.pad(W_ih_f.T, ((0, INP - IN), (0, 0)))
    wih_b = jnp.pad(W_ih_b.T, ((0, INP - IN), (0, 0)))
    whh_f = W_hh_f.T
    whh_b = W_hh_b.T

    out = pl.pallas_call(
        _fused_kernel,
        out_shape=jax.ShapeDtypeStruct((B, 1), jnp.float32),
        scratch_shapes=[pltpu.VMEM((N, B, INP), jnp.float32),
                        pltpu.VMEM((CH, B, G), jnp.float32),
                        pltpu.VMEM((CH, B, G), jnp.float32)],
    )(device_idx.reshape(B, 1), matrix, xaug,
      wih_f, whh_f, b_ih_f.reshape(1, G), b_hh_f.reshape(1, G),
      wih_b, whh_b, b_ih_b.reshape(1, G), b_hh_b.reshape(1, G),
      W_fc[:, 1:].T, W_fc[:, :1], b_fc.reshape(1, 1))
    return out.reshape(-1)
